# baseline (speedup 1.0000x reference)
import functools

import jax
import jax.numpy as jnp
from jax import lax
from jax.experimental import pallas as pl
from jax.experimental.pallas import tpu as pltpu
from jax.experimental.pallas import tpu_sc as plsc

_L = 16

_MESH = plsc.VectorSubcoreMesh(core_axis_name="c", subcore_axis_name="s")


@functools.partial(
    pl.kernel,
    out_type=jax.ShapeDtypeStruct((_L,), jnp.float32),
    mesh=_MESH,
    compiler_params=pltpu.CompilerParams(needs_layout_passes=False),
    scratch_types=[
        pltpu.VMEM((6 * _L,), jnp.float32),
        pltpu.VMEM((_L,), jnp.float32),
        pltpu.VMEM((_L,), jnp.int32),
        pltpu.VMEM((_L,), jnp.float32),
    ],
)
def _sc_row_dot(m_hbm, t_hbm, s_hbm, out_hbm, m_v, t_v, s_v, o_v):
    wid = lax.axis_index("s") * _MESH.num_cores + lax.axis_index("c")

    @pl.when(wid == 0)
    def _():
        pltpu.sync_copy(m_hbm, m_v)
        pltpu.sync_copy(t_hbm, t_v)
        pltpu.sync_copy(s_hbm, s_v)
        idx = s_v[...] * _L + lax.iota(jnp.int32, _L)
        row = plsc.load_gather(m_v, [idx])
        val = jnp.sum(row * t_v[...])
        o_v[...] = jnp.full((_L,), val, jnp.float32)
        pltpu.sync_copy(o_v, out_hbm)


def kernel(state, M, theta):
    m_pad = jnp.zeros((6, _L), jnp.float32).at[:, :7].set(M).reshape(6 * _L)
    t_pad = jnp.zeros((_L,), jnp.float32).at[:7].set(theta)
    s_vec = jnp.full((_L,), state, jnp.int32)
    out = _sc_row_dot(m_pad, t_pad, s_vec)
    return out[0]


# speedup vs baseline: 1.1508x; 1.1508x over previous
import functools

import jax
import jax.numpy as jnp
from jax import lax
from jax.experimental import pallas as pl
from jax.experimental.pallas import tpu as pltpu
from jax.experimental.pallas import tpu_sc as plsc

_L = 16

_MESH = plsc.VectorSubcoreMesh(
    core_axis_name="c", subcore_axis_name="s", num_cores=1, num_subcores=1
)


@functools.partial(
    pl.kernel,
    out_type=jax.ShapeDtypeStruct((1,), jnp.float32),
    mesh=_MESH,
    compiler_params=pltpu.CompilerParams(needs_layout_passes=False),
    scratch_types=[
        pltpu.VMEM((48,), jnp.float32),
        pltpu.VMEM((8,), jnp.float32),
        pltpu.VMEM((_L,), jnp.int32),
        pltpu.VMEM((_L,), jnp.float32),
    ],
)
def _sc_row_dot(m_hbm, t_hbm, s_hbm, out_hbm, m_v, t_v, s_v, o_v):
    pltpu.sync_copy(m_hbm, m_v.at[pl.ds(0, 42)])
    pltpu.sync_copy(t_hbm, t_v.at[pl.ds(0, 7)])
    pltpu.sync_copy(s_hbm, s_v.at[pl.ds(0, 1)])
    s = s_v[...][0]
    lanes = lax.iota(jnp.int32, _L)
    row = plsc.load_gather(m_v, [jnp.minimum(s * 7 + lanes, 41)])
    tv = plsc.load_gather(t_v, [jnp.minimum(lanes, 6)])
    prod = jnp.where(lanes < 7, row * tv, 0.0)
    o_v[...] = jnp.full((_L,), jnp.sum(prod), jnp.float32)
    pltpu.sync_copy(o_v.at[pl.ds(0, 1)], out_hbm)


def kernel(state, M, theta):
    s_arr = jnp.asarray(state, jnp.int32).reshape(1)
    out = _sc_row_dot(M.reshape(42), theta, s_arr)
    return out.reshape(())


# speedup vs baseline: 1.2132x; 1.0543x over previous
import functools

import jax
import jax.numpy as jnp
from jax import lax
from jax.experimental import pallas as pl
from jax.experimental.pallas import tpu as pltpu
from jax.experimental.pallas import tpu_sc as plsc

_L = 16

_MESH = plsc.VectorSubcoreMesh(
    core_axis_name="c", subcore_axis_name="s", num_cores=1, num_subcores=1
)


@functools.partial(
    pl.kernel,
    out_type=jax.ShapeDtypeStruct((1,), jnp.float32),
    mesh=_MESH,
    compiler_params=pltpu.CompilerParams(needs_layout_passes=False),
    scratch_types=[
        pltpu.VMEM((48,), jnp.float32),
        pltpu.VMEM((8,), jnp.float32),
        pltpu.VMEM((_L,), jnp.int32),
        pltpu.VMEM((_L,), jnp.float32),
        pltpu.SemaphoreType.DMA,
        pltpu.SemaphoreType.DMA,
        pltpu.SemaphoreType.DMA,
    ],
)
def _sc_row_dot(m_hbm, t_hbm, s_hbm, out_hbm, m_v, t_v, s_v, o_v,
                sem_m, sem_t, sem_s):
    cp_m = pltpu.async_copy(m_hbm, m_v.at[pl.ds(0, 42)], sem_m)
    cp_t = pltpu.async_copy(t_hbm, t_v.at[pl.ds(0, 7)], sem_t)
    cp_s = pltpu.async_copy(s_hbm, s_v.at[pl.ds(0, 1)], sem_s)
    cp_m.wait()
    cp_t.wait()
    cp_s.wait()
    s = s_v[...][0]
    lanes = lax.iota(jnp.int32, _L)
    row = plsc.load_gather(m_v, [jnp.minimum(s * 7 + lanes, 41)])
    tv = plsc.load_gather(t_v, [jnp.minimum(lanes, 6)])
    prod = jnp.where(lanes < 7, row * tv, 0.0)
    o_v[...] = jnp.full((_L,), jnp.sum(prod), jnp.float32)
    pltpu.sync_copy(o_v.at[pl.ds(0, 1)], out_hbm)


def kernel(state, M, theta):
    s_arr = jnp.asarray(state, jnp.int32).reshape(1)
    out = _sc_row_dot(M.reshape(42), theta, s_arr)
    return out.reshape(())
